# baseline (speedup 1.0000x reference)
import jax
import jax.numpy as jnp
from jax.experimental import pallas as pl
from jax.experimental.pallas import tpu as pltpu

_H = 384
_W = 384
_TILE = 128
_STRIDE = 32
_NT = 9
_TAPS = tuple((dy, dx) for dy in range(3) for dx in range(3))


def _tree(ts):
    ts = list(ts)
    while len(ts) > 1:
        nxt = [ts[n] + ts[n + 1] for n in range(0, len(ts) - 1, 2)]
        if len(ts) % 2:
            nxt.append(ts[-1])
        ts = nxt
    return ts[0]


def _reduce27(prods):
    acc = _tree(prods[0:8])
    for base in (8, 16, 24):
        acc = acc + _tree(prods[base:base + 8])
    return acc


def _body(xs_ref, w1_ref, b1_ref, w2_ref, b2_ref, w1m_ref, b1m_ref,
          head_ref, seg_ref, segc_ref, hs_ref, rt_ref, rb_ref):
    j = pl.program_id(0)
    i = pl.program_id(1)

    @pl.when((i == 0) & (j == 0))
    def _init():
        segc_ref[...] = jnp.zeros_like(segc_ref)

    a = j // 4
    k = j - 4 * a
    row = i * _STRIDE
    colb = a * _TILE

    @pl.when(i == 0)
    def _conv1_strip():
        xcol = xs_ref[k, :, :, pl.ds(colb, _TILE)].astype(jnp.float32)
        xp = jnp.pad(xcol, ((0, 0), (1, 1), (1, 1)))
        xw = jnp.stack([xp[ci, dy:dy + _H, dx:dx + _TILE]
                        for dy, dx in _TAPS for ci in range(3)])

        def c1_body(c, carry):
            prods = []
            t = 0
            for dy, dx in _TAPS:
                for ci in range(3):
                    prods.append(w1_ref[c, ci, dy, dx] * xw[t])
                    t += 1
            acc = _reduce27(prods) + b1_ref[c]
            hs_ref[c] = jnp.maximum(acc, 0.0).astype(jnp.bfloat16)
            return carry

        jax.lax.fori_loop(0, 32, c1_body, 0, unroll=False)

    xt2 = jnp.pad(
        xs_ref[k, :, pl.ds(row, 2), pl.ds(colb, _TILE)].astype(jnp.float32),
        ((0, 0), (0, 0), (1, 1)))
    xb2 = jnp.pad(
        xs_ref[k, :, pl.ds(row + 120, 8), pl.ds(colb, _TILE)][:, 6:8].astype(jnp.float32),
        ((0, 0), (0, 0), (1, 1)))
    zrow = jnp.zeros((32, _TILE), jnp.float32)

    def ring(xrows):
        prods = []
        t = 0
        for dy, dx in _TAPS:
            for ci in range(3):
                if xrows[dy] is None:
                    prods.append(zrow)
                else:
                    wv = w1m_ref[:, t:t + 1]
                    prods.append(wv * xrows[dy][ci, dx:dx + _TILE][None, :])
                t += 1
        acc = _reduce27(prods) + b1m_ref[...]
        return jnp.maximum(acc, 0.0).astype(jnp.bfloat16)

    @pl.when(i > 0)
    def _ring_top():
        rt_ref[...] = ring([None, xt2[:, 0], xt2[:, 1]])

    @pl.when(i < _NT - 1)
    def _ring_bot():
        rb_ref[...] = ring([xb2[:, 0], xb2[:, 1], None])

    wds = [[w2_ref[1, ci, dy, dx] - w2_ref[0, ci, dy, dx]
            for (dy, dx) in _TAPS] for ci in range(32)]
    riota = jax.lax.broadcasted_iota(jnp.int32, (_STRIDE, _TILE), 0)
    top_mask = (riota == 0) & (i > 0)
    bot_mask = (riota == _STRIDE - 1) & (i < _NT - 1)
    strips = []
    for s in range(4):
        accs = [jnp.zeros((_STRIDE, _TILE), jnp.float32) for _ in _TAPS]
        for ci in range(32):
            hstrip = hs_ref[ci, pl.ds(row + _STRIDE * s, _STRIDE), :].astype(jnp.float32)
            if s == 0:
                hstrip = jnp.where(top_mask, rt_ref[ci].astype(jnp.float32)[None, :], hstrip)
            if s == 3:
                hstrip = jnp.where(bot_mask, rb_ref[ci].astype(jnp.float32)[None, :], hstrip)
            for t in range(9):
                accs[t] = accs[t] + wds[ci][t] * hstrip
        strips.append(accs)

    out = jnp.full((_TILE, _TILE), b2_ref[1] - b2_ref[0], jnp.float32)
    for t, (dy, dx) in enumerate(_TAPS):
        ptap = jnp.concatenate([strips[s][t] for s in range(4)], axis=0)
        pp = jnp.pad(ptap, ((1, 1), (1, 1)))
        out = out + pp[dy:dy + _TILE, dx:dx + _TILE]

    cur = segc_ref[k, pl.ds(row, _TILE), pl.ds(colb, _TILE)]
    segc_ref[k, pl.ds(row, _TILE), pl.ds(colb, _TILE)] = cur + out

    @pl.when((i == _NT - 1) & (j == _NT - 1))
    def _head():
        s0 = segc_ref[0]
        for k2 in range(1, 4):
            sh = _STRIDE * k2
            s0 = s0 + jnp.pad(segc_ref[k2, :, :_W - sh], ((0, 0), (sh, 0)))
        seg_ref[...] = s0
        sp = jnp.maximum(s0, 0.0)
        spp = jnp.pad(sp, ((1, 1), (1, 1)))
        m = jnp.zeros_like(s0)
        for dy in range(3):
            for dx in range(3):
                if dy == 1 and dx == 1:
                    continue
                m = jnp.maximum(m, spp[dy:dy + _H, dx:dx + _W])
        head_ref[0] = jnp.where(s0 > m, sp, 0.0)


def kernel(x, W1, b1, W2, b2):
    x = x.astype(jnp.bfloat16)
    W1 = jax.lax.optimization_barrier(W1.astype(jnp.bfloat16)).astype(jnp.float32)
    W2 = jax.lax.optimization_barrier(W2.astype(jnp.bfloat16)).astype(jnp.float32)
    xs = jnp.stack([
        jnp.pad(x[:, :, 32 * k:], ((0, 0), (0, 0), (0, 32 * k)))
        for k in range(4)
    ])
    W1m = jnp.transpose(W1, (0, 2, 3, 1)).reshape(32, 27)
    b1m = b1[:, None]
    head, seg = pl.pallas_call(
        _body,
        grid=(_NT, _NT),
        in_specs=[
            pl.BlockSpec((4, 3, _H, _W), lambda j, i: (0, 0, 0, 0)),
            pl.BlockSpec(memory_space=pltpu.SMEM),
            pl.BlockSpec(memory_space=pltpu.SMEM),
            pl.BlockSpec(memory_space=pltpu.SMEM),
            pl.BlockSpec(memory_space=pltpu.SMEM),
            pl.BlockSpec((32, 27), lambda j, i: (0, 0)),
            pl.BlockSpec((32, 1), lambda j, i: (0, 0)),
        ],
        out_specs=[
            pl.BlockSpec((1, _H, _W), lambda j, i: (0, 0, 0)),
            pl.BlockSpec((_H, _W), lambda j, i: (0, 0)),
        ],
        out_shape=[
            jax.ShapeDtypeStruct((1, _H, _W), jnp.float32),
            jax.ShapeDtypeStruct((_H, _W), jnp.float32),
        ],
        scratch_shapes=[
            pltpu.VMEM((4, _H, _W), jnp.float32),
            pltpu.VMEM((32, _H, _TILE), jnp.bfloat16),
            pltpu.VMEM((32, _TILE), jnp.bfloat16),
            pltpu.VMEM((32, _TILE), jnp.bfloat16),
        ],
        compiler_params=pltpu.CompilerParams(
            dimension_semantics=("arbitrary", "arbitrary"),
        ),
    )(xs, W1, b1, W2, b2, W1m, b1m)
    return (head, seg)


# speedup vs baseline: 1.0807x; 1.0807x over previous
import jax
import jax.numpy as jnp
from jax.experimental import pallas as pl
from jax.experimental.pallas import tpu as pltpu

_H = 384
_W = 384
_TILE = 128
_STRIDE = 32
_NT = 9
_TAPS = tuple((dy, dx) for dy in range(3) for dx in range(3))


def _tree(ts):
    ts = list(ts)
    while len(ts) > 1:
        nxt = [ts[n] + ts[n + 1] for n in range(0, len(ts) - 1, 2)]
        if len(ts) % 2:
            nxt.append(ts[-1])
        ts = nxt
    return ts[0]


def _reduce27(prods):
    acc = _tree(prods[0:8])
    for base in (8, 16, 24):
        acc = acc + _tree(prods[base:base + 8])
    return acc


def _body(xs_ref, w1_ref, b1_ref, w2_ref, b2_ref, w1m_ref, b1m_ref,
          head_ref, seg_ref, segc_ref, hs_ref, rt_ref, rb_ref):
    j = pl.program_id(0)
    i = pl.program_id(1)

    @pl.when((i == 0) & (j == 0))
    def _init():
        segc_ref[...] = jnp.zeros_like(segc_ref)

    a = j // 4
    k = j - 4 * a
    row = i * _STRIDE
    colb = a * _TILE

    @pl.when(i == 0)
    def _conv1_strip():
        xcol = xs_ref[k, :, :, pl.ds(colb, _TILE)].astype(jnp.float32)
        xp = jnp.pad(xcol, ((0, 0), (1, 1), (1, 1)))
        xw = jnp.stack([xp[ci, dy:dy + _H, dx:dx + _TILE]
                        for dy, dx in _TAPS for ci in range(3)])

        def c1_body(c, carry):
            prods = []
            t = 0
            for dy, dx in _TAPS:
                for ci in range(3):
                    prods.append(w1_ref[c, ci, dy, dx] * xw[t])
                    t += 1
            acc = _reduce27(prods) + b1_ref[c]
            hs_ref[c] = jnp.maximum(acc, 0.0).astype(jnp.bfloat16)
            return carry

        jax.lax.fori_loop(0, 32, c1_body, 0, unroll=False)

    xt2 = jnp.pad(
        xs_ref[k, :, pl.ds(row, 2), pl.ds(colb, _TILE)].astype(jnp.float32),
        ((0, 0), (0, 0), (1, 1)))
    xb2 = jnp.pad(
        xs_ref[k, :, pl.ds(row + 120, 8), pl.ds(colb, _TILE)][:, 6:8].astype(jnp.float32),
        ((0, 0), (0, 0), (1, 1)))
    zrow = jnp.zeros((32, _TILE), jnp.float32)

    def ring(xrows):
        prods = []
        t = 0
        for dy, dx in _TAPS:
            for ci in range(3):
                if xrows[dy] is None:
                    prods.append(zrow)
                else:
                    wv = w1m_ref[:, t:t + 1]
                    prods.append(wv * xrows[dy][ci, dx:dx + _TILE][None, :])
                t += 1
        acc = _reduce27(prods) + b1m_ref[...]
        return jnp.maximum(acc, 0.0).astype(jnp.bfloat16)

    rt_ref[...] = ring([None, xt2[:, 0], xt2[:, 1]])
    rb_ref[...] = ring([xb2[:, 0], xb2[:, 1], None])

    wds = [[w2_ref[1, ci, dy, dx] - w2_ref[0, ci, dy, dx]
            for (dy, dx) in _TAPS] for ci in range(32)]
    riota = jax.lax.broadcasted_iota(jnp.int32, (_STRIDE, _TILE), 0)
    top_mask = (riota == 0) & (i > 0)
    bot_mask = (riota == _STRIDE - 1) & (i < _NT - 1)
    strips = []
    for s in range(4):
        accs = [jnp.zeros((_STRIDE, _TILE), jnp.float32) for _ in _TAPS]
        for ci in range(32):
            hstrip = hs_ref[ci, pl.ds(row + _STRIDE * s, _STRIDE), :].astype(jnp.float32)
            if s == 0:
                hstrip = jnp.where(top_mask, rt_ref[ci].astype(jnp.float32)[None, :], hstrip)
            if s == 3:
                hstrip = jnp.where(bot_mask, rb_ref[ci].astype(jnp.float32)[None, :], hstrip)
            for t in range(9):
                accs[t] = accs[t] + wds[ci][t] * hstrip
        strips.append(accs)

    out = jnp.full((_TILE, _TILE), b2_ref[1] - b2_ref[0], jnp.float32)
    for t, (dy, dx) in enumerate(_TAPS):
        ptap = jnp.concatenate([strips[s][t] for s in range(4)], axis=0)
        pp = jnp.pad(ptap, ((1, 1), (1, 1)))
        out = out + pp[dy:dy + _TILE, dx:dx + _TILE]

    cur = segc_ref[k, pl.ds(row, _TILE), pl.ds(colb, _TILE)]
    segc_ref[k, pl.ds(row, _TILE), pl.ds(colb, _TILE)] = cur + out

    @pl.when((i == _NT - 1) & (j == _NT - 1))
    def _head():
        s0 = segc_ref[0]
        for k2 in range(1, 4):
            sh = _STRIDE * k2
            s0 = s0 + jnp.pad(segc_ref[k2, :, :_W - sh], ((0, 0), (sh, 0)))
        seg_ref[...] = s0
        sp = jnp.maximum(s0, 0.0)
        spp = jnp.pad(sp, ((1, 1), (1, 1)))
        m = jnp.zeros_like(s0)
        for dy in range(3):
            for dx in range(3):
                if dy == 1 and dx == 1:
                    continue
                m = jnp.maximum(m, spp[dy:dy + _H, dx:dx + _W])
        head_ref[0] = jnp.where(s0 > m, sp, 0.0)


def kernel(x, W1, b1, W2, b2):
    x = x.astype(jnp.bfloat16)
    W1 = jax.lax.optimization_barrier(W1.astype(jnp.bfloat16)).astype(jnp.float32)
    W2 = jax.lax.optimization_barrier(W2.astype(jnp.bfloat16)).astype(jnp.float32)
    xs = jnp.stack([
        jnp.pad(x[:, :, 32 * k:], ((0, 0), (0, 0), (0, 32 * k)))
        for k in range(4)
    ])
    W1m = jnp.transpose(W1, (0, 2, 3, 1)).reshape(32, 27)
    b1m = b1[:, None]
    head, seg = pl.pallas_call(
        _body,
        grid=(_NT, _NT),
        in_specs=[
            pl.BlockSpec((4, 3, _H, _W), lambda j, i: (0, 0, 0, 0)),
            pl.BlockSpec(memory_space=pltpu.SMEM),
            pl.BlockSpec(memory_space=pltpu.SMEM),
            pl.BlockSpec(memory_space=pltpu.SMEM),
            pl.BlockSpec(memory_space=pltpu.SMEM),
            pl.BlockSpec((32, 27), lambda j, i: (0, 0)),
            pl.BlockSpec((32, 1), lambda j, i: (0, 0)),
        ],
        out_specs=[
            pl.BlockSpec((1, _H, _W), lambda j, i: (0, 0, 0)),
            pl.BlockSpec((_H, _W), lambda j, i: (0, 0)),
        ],
        out_shape=[
            jax.ShapeDtypeStruct((1, _H, _W), jnp.float32),
            jax.ShapeDtypeStruct((_H, _W), jnp.float32),
        ],
        scratch_shapes=[
            pltpu.VMEM((4, _H, _W), jnp.float32),
            pltpu.VMEM((32, _H, _TILE), jnp.bfloat16),
            pltpu.VMEM((32, _TILE), jnp.bfloat16),
            pltpu.VMEM((32, _TILE), jnp.bfloat16),
        ],
        compiler_params=pltpu.CompilerParams(
            dimension_semantics=("arbitrary", "arbitrary"),
        ),
    )(xs, W1, b1, W2, b2, W1m, b1m)
    return (head, seg)
